# baseline (speedup 1.0000x reference)
import functools

import jax
import jax.numpy as jnp
from jax import lax
from jax.experimental import pallas as pl
from jax.experimental.pallas import tpu as pltpu
from jax.experimental.pallas import tpu_sc as plsc

H = 64
HP = H // 2
C = 128
L = 16
NC = 2
NS = 16
NW = NC * NS
N_REL = 100


@functools.partial(jax.jit, static_argnames=("npw",))
def _sc_score(emb, wrel, idx_all, npw):
    n_chunks = idx_all.shape[0]
    np_total = n_chunks * C
    mesh = plsc.VectorSubcoreMesh(core_axis_name="c", subcore_axis_name="s")

    @functools.partial(
        pl.kernel,
        mesh=mesh,
        compiler_params=pltpu.CompilerParams(
            needs_layout_passes=False, use_tc_tiling_on_sc=False),
        out_type=jax.ShapeDtypeStruct((np_total,), jnp.float32),
        scratch_types=[
            pltpu.VMEM((N_REL, HP), jnp.int32),
            pltpu.VMEM((2, 3, C), jnp.int32),
            pltpu.VMEM((2, C, HP), jnp.int32),
            pltpu.VMEM((2, C, HP), jnp.int32),
            pltpu.VMEM((2, C), jnp.float32),
            pltpu.SemaphoreType.DMA,
            pltpu.SemaphoreType.DMA,
            pltpu.SemaphoreType.DMA,
            pltpu.SemaphoreType.DMA,
            pltpu.SemaphoreType.DMA,
            pltpu.SemaphoreType.DMA,
        ],
    )
    def k(emb_h, wrel_h, idx_h, out_h,
          wrel_v, idx_v, s_v, o_v, out_v,
          semi0, semi1, semr0, semr1, semo0, semo1):
        wid = lax.axis_index("s") * NC + lax.axis_index("c")
        base_chunk = wid * npw
        semi = (semi0, semi1)
        semr = (semr0, semr1)
        semo = (semo0, semo1)

        pltpu.sync_copy(wrel_h, wrel_v)

        def issue_idx(t, b):
            pltpu.async_copy(idx_h.at[base_chunk + t], idx_v.at[b], semi[b])

        def wait_idx(b):
            pltpu.make_async_copy(idx_h.at[0], idx_v.at[b], semi[b]).wait()

        def issue_rows(b):
            pltpu.async_copy(emb_h.at[idx_v.at[b, 0]], s_v.at[b], semr[b])
            pltpu.async_copy(emb_h.at[idx_v.at[b, 2]], o_v.at[b], semr[b])

        def wait_rows(b):
            pltpu.make_async_copy(emb_h.at[idx_v.at[b, 0]], s_v.at[b],
                                  semr[b]).wait()
            pltpu.make_async_copy(emb_h.at[idx_v.at[b, 2]], o_v.at[b],
                                  semr[b]).wait()

        def store_out(t, b):
            off = (base_chunk + t) * C
            pltpu.async_copy(out_v.at[b], out_h.at[pl.ds(off, C)], semo[b])

        def wait_out(t, b):
            off = (base_chunk + t) * C
            pltpu.make_async_copy(out_v.at[b], out_h.at[pl.ds(off, C)],
                                  semo[b]).wait()

        himask = jnp.full((L,), -65536, jnp.int32)

        def unpack2(x):
            lo = plsc.bitcast(lax.shift_left(x, 16), jnp.float32)
            hi = plsc.bitcast(lax.bitwise_and(x, himask), jnp.float32)
            return lo, hi

        def compute(b):
            def iblk(i0, _):
                rows = i0 * L + lax.iota(jnp.int32, L)
                relv = idx_v[b, 1, pl.ds(i0 * L, L)]
                lane = lax.iota(jnp.int32, L)
                zero = jnp.zeros((L,), jnp.float32)

                def dgrp(g, accs):
                    accs = list(accs)
                    for dd in range(16):
                        d = g * 16 + dd
                        cols = (lane + d) & (HP - 1)
                        sp = plsc.load_gather(s_v.at[b], [rows, cols])
                        op_ = plsc.load_gather(o_v.at[b], [rows, cols])
                        rp = plsc.load_gather(wrel_v, [relv, cols])
                        slo, shi = unpack2(sp)
                        olo, ohi = unpack2(op_)
                        rlo, rhi = unpack2(rp)
                        j = dd % 2
                        accs[j] = accs[j] + slo * olo * rlo
                        accs[2 + j] = accs[2 + j] + shi * ohi * rhi
                    return tuple(accs)

                accs = lax.fori_loop(0, HP // 16, dgrp,
                                     (zero, zero, zero, zero))
                out_v[b, pl.ds(i0 * L, L)] = (
                    (accs[0] + accs[1]) + (accs[2] + accs[3]))
                return _

            lax.fori_loop(0, C // L, iblk, None)

        issue_idx(0, 0)
        issue_idx(1, 1)
        wait_idx(0)
        issue_rows(0)

        def step(t, b):
            wait_rows(b)

            @pl.when(t + 1 < npw)
            def _():
                wait_idx(1 - b)
                issue_rows(1 - b)

            @pl.when(t >= 2)
            def _():
                wait_out(t - 2, b)

            compute(b)
            @pl.when(t + 2 < npw)
            def _():
                issue_idx(t + 2, b)

            store_out(t, b)

        def outer(g, _):
            step(g * 2, 0)
            step(g * 2 + 1, 1)
            return _

        lax.fori_loop(0, npw // 2, outer, None)
        wait_out(npw - 2, 0)
        wait_out(npw - 1, 1)

    return k(emb, wrel, idx_all)


def _pack_bf16(table):
    b = table.astype(jnp.bfloat16).reshape(table.shape[0], HP, 2)
    return lax.bitcast_convert_type(b, jnp.int32)


def kernel(embedding, w_relation, triplets):
    n = triplets.shape[0]
    n_chunks = -(-n // C)
    npw = -(-n_chunks // NW)
    n_chunks = NW * npw
    np_total = n_chunks * C
    trip = jnp.pad(triplets.astype(jnp.int32), ((0, np_total - n), (0, 0)))
    idx_all = trip.reshape(n_chunks, C, 3).transpose(0, 2, 1)
    out = _sc_score(_pack_bf16(embedding), _pack_bf16(w_relation),
                    idx_all, npw)
    return out[:n]


# speedup vs baseline: 1.0097x; 1.0097x over previous
import functools

import jax
import jax.numpy as jnp
from jax import lax
from jax.experimental import pallas as pl
from jax.experimental.pallas import tpu as pltpu
from jax.experimental.pallas import tpu_sc as plsc

H = 64
HP = H // 2
C = 128
L = 16
NC = 2
NS = 16
NW = NC * NS
N_REL = 100


@functools.partial(jax.jit, static_argnames=("npw",))
def _sc_score(emb, wrel, idx_all, npw):
    n_chunks = idx_all.shape[0]
    np_total = n_chunks * C
    mesh = plsc.VectorSubcoreMesh(core_axis_name="c", subcore_axis_name="s")

    @functools.partial(
        pl.kernel,
        mesh=mesh,
        compiler_params=pltpu.CompilerParams(
            needs_layout_passes=False, use_tc_tiling_on_sc=False),
        out_type=jax.ShapeDtypeStruct((np_total,), jnp.float32),
        scratch_types=[
            pltpu.VMEM((N_REL, HP), jnp.int32),
            pltpu.VMEM((2, 3, C), jnp.int32),
            pltpu.VMEM((2, C, HP), jnp.int32),
            pltpu.VMEM((2, C, HP), jnp.int32),
            pltpu.VMEM((2, C), jnp.float32),
            pltpu.SemaphoreType.DMA,
            pltpu.SemaphoreType.DMA,
            pltpu.SemaphoreType.DMA,
            pltpu.SemaphoreType.DMA,
            pltpu.SemaphoreType.DMA,
            pltpu.SemaphoreType.DMA,
        ],
    )
    def k(emb_h, wrel_h, idx_h, out_h,
          wrel_v, idx_v, s_v, o_v, out_v,
          semi0, semi1, semr0, semr1, semo0, semo1):
        wid = lax.axis_index("s") * NC + lax.axis_index("c")
        base_chunk = wid * npw
        semi = (semi0, semi1)
        semr = (semr0, semr1)
        semo = (semo0, semo1)

        pltpu.sync_copy(wrel_h, wrel_v)

        def issue_idx(t, b):
            pltpu.async_copy(idx_h.at[base_chunk + t], idx_v.at[b], semi[b])

        def wait_idx(b):
            pltpu.make_async_copy(idx_h.at[0], idx_v.at[b], semi[b]).wait()

        def issue_rows(b):
            pltpu.async_copy(emb_h.at[idx_v.at[b, 0]], s_v.at[b], semr[b])
            pltpu.async_copy(emb_h.at[idx_v.at[b, 2]], o_v.at[b], semr[b])

        def wait_rows(b):
            pltpu.make_async_copy(emb_h.at[idx_v.at[b, 0]], s_v.at[b],
                                  semr[b]).wait()
            pltpu.make_async_copy(emb_h.at[idx_v.at[b, 2]], o_v.at[b],
                                  semr[b]).wait()

        def store_out(t, b):
            off = (base_chunk + t) * C
            pltpu.async_copy(out_v.at[b], out_h.at[pl.ds(off, C)], semo[b])

        def wait_out(t, b):
            off = (base_chunk + t) * C
            pltpu.make_async_copy(out_v.at[b], out_h.at[pl.ds(off, C)],
                                  semo[b]).wait()

        himask = jnp.full((L,), -65536, jnp.int32)

        def unpack2(x):
            lo = plsc.bitcast(lax.shift_left(x, 16), jnp.float32)
            hi = plsc.bitcast(lax.bitwise_and(x, himask), jnp.float32)
            return lo, hi

        def compute(b):
            @plsc.parallel_loop(0, C // L, 1, unroll=1)
            def iblk(i0):
                rows = i0 * L + lax.iota(jnp.int32, L)
                relv = idx_v[b, 1, pl.ds(i0 * L, L)]
                lane = lax.iota(jnp.int32, L)
                zero = jnp.zeros((L,), jnp.float32)

                def dgrp(g, accs):
                    accs = list(accs)
                    for dd in range(8):
                        d = g * 8 + dd
                        cols = (lane + d) & (HP - 1)
                        sp = plsc.load_gather(s_v.at[b], [rows, cols])
                        op_ = plsc.load_gather(o_v.at[b], [rows, cols])
                        rp = plsc.load_gather(wrel_v, [relv, cols])
                        slo, shi = unpack2(sp)
                        olo, ohi = unpack2(op_)
                        rlo, rhi = unpack2(rp)
                        j = dd % 2
                        accs[j] = accs[j] + slo * olo * rlo
                        accs[2 + j] = accs[2 + j] + shi * ohi * rhi
                    return tuple(accs)

                accs = lax.fori_loop(0, HP // 8, dgrp,
                                     (zero, zero, zero, zero))
                out_v[b, pl.ds(i0 * L, L)] = (
                    (accs[0] + accs[1]) + (accs[2] + accs[3]))

        issue_idx(0, 0)
        issue_idx(1, 1)
        wait_idx(0)
        issue_rows(0)

        def step(t, b):
            wait_rows(b)

            @pl.when(t + 1 < npw)
            def _():
                wait_idx(1 - b)
                issue_rows(1 - b)

            @pl.when(t >= 2)
            def _():
                wait_out(t - 2, b)

            compute(b)
            @pl.when(t + 2 < npw)
            def _():
                issue_idx(t + 2, b)

            store_out(t, b)

        def outer(g, _):
            step(g * 2, 0)
            step(g * 2 + 1, 1)
            return _

        lax.fori_loop(0, npw // 2, outer, None)
        wait_out(npw - 2, 0)
        wait_out(npw - 1, 1)

    return k(emb, wrel, idx_all)


def _pack_bf16(table):
    b = table.astype(jnp.bfloat16).reshape(table.shape[0], HP, 2)
    return lax.bitcast_convert_type(b, jnp.int32)


def kernel(embedding, w_relation, triplets):
    n = triplets.shape[0]
    n_chunks = -(-n // C)
    npw = -(-n_chunks // NW)
    n_chunks = NW * npw
    np_total = n_chunks * C
    trip = jnp.pad(triplets.astype(jnp.int32), ((0, np_total - n), (0, 0)))
    idx_all = trip.reshape(n_chunks, C, 3).transpose(0, 2, 1)
    out = _sc_score(_pack_bf16(embedding), _pack_bf16(w_relation),
                    idx_all, npw)
    return out[:n]


# speedup vs baseline: 1.0232x; 1.0133x over previous
import functools

import jax
import jax.numpy as jnp
from jax import lax
from jax.experimental import pallas as pl
from jax.experimental.pallas import tpu as pltpu
from jax.experimental.pallas import tpu_sc as plsc

H = 64
HP = H // 2
C = 128
L = 16
NC = 2
NS = 16
NW = NC * NS
N_REL = 100


@functools.partial(jax.jit, static_argnames=("npw",))
def _sc_score(emb, wrel, idx_all, npw):
    n_chunks = idx_all.shape[0]
    np_total = n_chunks * C
    mesh = plsc.VectorSubcoreMesh(core_axis_name="c", subcore_axis_name="s")

    @functools.partial(
        pl.kernel,
        mesh=mesh,
        compiler_params=pltpu.CompilerParams(
            needs_layout_passes=False, use_tc_tiling_on_sc=False),
        out_type=jax.ShapeDtypeStruct((np_total,), jnp.float32),
        scratch_types=[
            pltpu.VMEM((N_REL, HP), jnp.int32),
            pltpu.VMEM((2, 3, C), jnp.int32),
            pltpu.VMEM((2, C, HP), jnp.int32),
            pltpu.VMEM((2, C, HP), jnp.int32),
            pltpu.VMEM((2, C), jnp.float32),
            pltpu.SemaphoreType.DMA,
            pltpu.SemaphoreType.DMA,
            pltpu.SemaphoreType.DMA,
            pltpu.SemaphoreType.DMA,
            pltpu.SemaphoreType.DMA,
            pltpu.SemaphoreType.DMA,
        ],
    )
    def k(emb_h, wrel_h, idx_h, out_h,
          wrel_v, idx_v, s_v, o_v, out_v,
          semi0, semi1, semr0, semr1, semo0, semo1):
        wid = lax.axis_index("s") * NC + lax.axis_index("c")
        base_chunk = wid * npw
        semi = (semi0, semi1)
        semr = (semr0, semr1)
        semo = (semo0, semo1)

        pltpu.sync_copy(wrel_h, wrel_v)

        def issue_idx(t, b):
            pltpu.async_copy(idx_h.at[base_chunk + t], idx_v.at[b], semi[b])

        def wait_idx(b):
            pltpu.make_async_copy(idx_h.at[0], idx_v.at[b], semi[b]).wait()

        def issue_rows(b):
            pltpu.async_copy(emb_h.at[idx_v.at[b, 0]], s_v.at[b], semr[b])
            pltpu.async_copy(emb_h.at[idx_v.at[b, 2]], o_v.at[b], semr[b])

        def wait_rows(b):
            pltpu.make_async_copy(emb_h.at[idx_v.at[b, 0]], s_v.at[b],
                                  semr[b]).wait()
            pltpu.make_async_copy(emb_h.at[idx_v.at[b, 2]], o_v.at[b],
                                  semr[b]).wait()

        def store_out(t, b):
            off = (base_chunk + t) * C
            pltpu.async_copy(out_v.at[b], out_h.at[pl.ds(off, C)], semo[b])

        def wait_out(t, b):
            off = (base_chunk + t) * C
            pltpu.make_async_copy(out_v.at[b], out_h.at[pl.ds(off, C)],
                                  semo[b]).wait()

        himask = jnp.full((L,), -65536, jnp.int32)

        def unpack2(x):
            lo = plsc.bitcast(lax.shift_left(x, 16), jnp.float32)
            hi = plsc.bitcast(lax.bitwise_and(x, himask), jnp.float32)
            return lo, hi

        def compute(b):
            @plsc.parallel_loop(0, C // L, 1, unroll=2)
            def iblk(i0):
                rows = i0 * L + lax.iota(jnp.int32, L)
                relv = idx_v[b, 1, pl.ds(i0 * L, L)]
                lane = lax.iota(jnp.int32, L)
                zero = jnp.zeros((L,), jnp.float32)

                def dgrp(g, accs):
                    accs = list(accs)
                    for dd in range(8):
                        d = g * 8 + dd
                        cols = (lane + d) & (HP - 1)
                        sp = plsc.load_gather(s_v.at[b], [rows, cols])
                        op_ = plsc.load_gather(o_v.at[b], [rows, cols])
                        rp = plsc.load_gather(wrel_v, [relv, cols])
                        slo, shi = unpack2(sp)
                        olo, ohi = unpack2(op_)
                        rlo, rhi = unpack2(rp)
                        j = dd % 2
                        accs[j] = accs[j] + slo * olo * rlo
                        accs[2 + j] = accs[2 + j] + shi * ohi * rhi
                    return tuple(accs)

                accs = lax.fori_loop(0, HP // 8, dgrp,
                                     (zero, zero, zero, zero))
                out_v[b, pl.ds(i0 * L, L)] = (
                    (accs[0] + accs[1]) + (accs[2] + accs[3]))

        issue_idx(0, 0)
        issue_idx(1, 1)
        wait_idx(0)
        issue_rows(0)

        def step(t, b):
            wait_rows(b)

            @pl.when(t + 1 < npw)
            def _():
                wait_idx(1 - b)
                issue_rows(1 - b)

            @pl.when(t >= 2)
            def _():
                wait_out(t - 2, b)

            compute(b)
            @pl.when(t + 2 < npw)
            def _():
                issue_idx(t + 2, b)

            store_out(t, b)

        def outer(g, _):
            step(g * 2, 0)
            step(g * 2 + 1, 1)
            return _

        lax.fori_loop(0, npw // 2, outer, None)
        wait_out(npw - 2, 0)
        wait_out(npw - 1, 1)

    return k(emb, wrel, idx_all)


def _pack_bf16(table):
    b = table.astype(jnp.bfloat16).reshape(table.shape[0], HP, 2)
    return lax.bitcast_convert_type(b, jnp.int32)


def kernel(embedding, w_relation, triplets):
    n = triplets.shape[0]
    n_chunks = -(-n // C)
    npw = -(-n_chunks // NW)
    n_chunks = NW * npw
    np_total = n_chunks * C
    trip = jnp.pad(triplets.astype(jnp.int32), ((0, np_total - n), (0, 0)))
    idx_all = trip.reshape(n_chunks, C, 3).transpose(0, 2, 1)
    out = _sc_score(_pack_bf16(embedding), _pack_bf16(w_relation),
                    idx_all, npw)
    return out[:n]


# speedup vs baseline: 1.0399x; 1.0163x over previous
import functools

import jax
import jax.numpy as jnp
from jax import lax
from jax.experimental import pallas as pl
from jax.experimental.pallas import tpu as pltpu
from jax.experimental.pallas import tpu_sc as plsc

H = 64
HP = H // 2
C = 128
L = 16
NC = 2
NS = 16
NW = NC * NS
N_REL = 100


@functools.partial(jax.jit, static_argnames=("npw",))
def _sc_score(emb, wrel, idx_all, npw):
    n_chunks = idx_all.shape[0]
    np_total = n_chunks * C
    mesh = plsc.VectorSubcoreMesh(core_axis_name="c", subcore_axis_name="s")

    @functools.partial(
        pl.kernel,
        mesh=mesh,
        compiler_params=pltpu.CompilerParams(
            needs_layout_passes=False, use_tc_tiling_on_sc=False),
        out_type=jax.ShapeDtypeStruct((np_total,), jnp.float32),
        scratch_types=[
            pltpu.VMEM((N_REL, HP), jnp.int32),
            pltpu.VMEM((4, 3, C), jnp.int32),
            pltpu.VMEM((4, C, HP), jnp.int32),
            pltpu.VMEM((4, C, HP), jnp.int32),
            pltpu.VMEM((4, C), jnp.float32),
        ] + [pltpu.SemaphoreType.DMA] * 12,
    )
    def k(emb_h, wrel_h, idx_h, out_h,
          wrel_v, idx_v, s_v, o_v, out_v,
          semi0, semi1, semi2, semi3, semr0, semr1, semr2, semr3,
          semo0, semo1, semo2, semo3):
        wid = lax.axis_index("s") * NC + lax.axis_index("c")
        base_chunk = wid * npw
        semi = (semi0, semi1, semi2, semi3)
        semr = (semr0, semr1, semr2, semr3)
        semo = (semo0, semo1, semo2, semo3)

        pltpu.sync_copy(wrel_h, wrel_v)

        def issue_idx(t, b):
            pltpu.async_copy(idx_h.at[base_chunk + t], idx_v.at[b], semi[b])

        def wait_idx(b):
            pltpu.make_async_copy(idx_h.at[0], idx_v.at[b], semi[b]).wait()

        def issue_rows(b):
            pltpu.async_copy(emb_h.at[idx_v.at[b, 0]], s_v.at[b], semr[b])
            pltpu.async_copy(emb_h.at[idx_v.at[b, 2]], o_v.at[b], semr[b])

        def wait_rows(b):
            pltpu.make_async_copy(emb_h.at[idx_v.at[b, 0]], s_v.at[b],
                                  semr[b]).wait()
            pltpu.make_async_copy(emb_h.at[idx_v.at[b, 2]], o_v.at[b],
                                  semr[b]).wait()

        def store_out(t, b):
            off = (base_chunk + t) * C
            pltpu.async_copy(out_v.at[b], out_h.at[pl.ds(off, C)], semo[b])

        def wait_out(t, b):
            off = (base_chunk + t) * C
            pltpu.make_async_copy(out_v.at[b], out_h.at[pl.ds(off, C)],
                                  semo[b]).wait()

        himask = jnp.full((L,), -65536, jnp.int32)

        def unpack2(x):
            lo = plsc.bitcast(lax.shift_left(x, 16), jnp.float32)
            hi = plsc.bitcast(lax.bitwise_and(x, himask), jnp.float32)
            return lo, hi

        def compute(b):
            @plsc.parallel_loop(0, C // L, 1, unroll=2)
            def iblk(i0):
                rows = i0 * L + lax.iota(jnp.int32, L)
                relv = idx_v[b, 1, pl.ds(i0 * L, L)]
                lane = lax.iota(jnp.int32, L)
                zero = jnp.zeros((L,), jnp.float32)

                def dgrp(g, accs):
                    accs = list(accs)
                    for dd in range(8):
                        d = g * 8 + dd
                        cols = (lane + d) & (HP - 1)
                        sp = plsc.load_gather(s_v.at[b], [rows, cols])
                        op_ = plsc.load_gather(o_v.at[b], [rows, cols])
                        rp = plsc.load_gather(wrel_v, [relv, cols])
                        slo, shi = unpack2(sp)
                        olo, ohi = unpack2(op_)
                        rlo, rhi = unpack2(rp)
                        j = dd % 2
                        accs[j] = accs[j] + slo * olo * rlo
                        accs[2 + j] = accs[2 + j] + shi * ohi * rhi
                    return tuple(accs)

                accs = lax.fori_loop(0, HP // 8, dgrp,
                                     (zero, zero, zero, zero))
                out_v[b, pl.ds(i0 * L, L)] = (
                    (accs[0] + accs[1]) + (accs[2] + accs[3]))

        for j in range(4):
            issue_idx(j, j)
        for j in range(3):
            wait_idx(j)
            issue_rows(j)

        def step(t, b):
            wait_rows(b)

            @pl.when(t + 3 < npw)
            def _():
                wait_idx((b + 3) % 4)
                issue_rows((b + 3) % 4)

            @pl.when(t >= 4)
            def _():
                wait_out(t - 4, b)

            compute(b)
            @pl.when(t + 4 < npw)
            def _():
                issue_idx(t + 4, b)

            store_out(t, b)

        def outer(g, _):
            for j in range(4):
                step(g * 4 + j, j)
            return _

        lax.fori_loop(0, npw // 4, outer, None)
        for j in range(4):
            wait_out(npw - 4 + j, j)

    return k(emb, wrel, idx_all)


def _pack_bf16(table):
    b = table.astype(jnp.bfloat16).reshape(table.shape[0], HP, 2)
    return lax.bitcast_convert_type(b, jnp.int32)


def kernel(embedding, w_relation, triplets):
    n = triplets.shape[0]
    n_chunks = -(-n // C)
    npw = -(-n_chunks // NW)
    n_chunks = NW * npw
    np_total = n_chunks * C
    trip = jnp.pad(triplets.astype(jnp.int32), ((0, np_total - n), (0, 0)))
    idx_all = trip.reshape(n_chunks, C, 3).transpose(0, 2, 1)
    out = _sc_score(_pack_bf16(embedding), _pack_bf16(w_relation),
                    idx_all, npw)
    return out[:n]


# speedup vs baseline: 1.0414x; 1.0015x over previous
import functools

import jax
import jax.numpy as jnp
from jax import lax
from jax.experimental import pallas as pl
from jax.experimental.pallas import tpu as pltpu
from jax.experimental.pallas import tpu_sc as plsc

H = 64
HP = H // 2
C = 128
L = 16
NC = 2
NS = 16
NW = NC * NS
N_REL = 100


@functools.partial(jax.jit, static_argnames=("npw",))
def _sc_score(emb, wrel, idx_all, npw):
    n_chunks = idx_all.shape[0]
    np_total = n_chunks * C
    mesh = plsc.VectorSubcoreMesh(core_axis_name="c", subcore_axis_name="s")

    @functools.partial(
        pl.kernel,
        mesh=mesh,
        compiler_params=pltpu.CompilerParams(
            needs_layout_passes=False, use_tc_tiling_on_sc=False),
        out_type=jax.ShapeDtypeStruct((np_total,), jnp.float32),
        scratch_types=[
            pltpu.VMEM((N_REL, HP), jnp.int32),
            pltpu.VMEM((4, 3, C), jnp.int32),
            pltpu.VMEM((4, C, HP), jnp.int32),
            pltpu.VMEM((4, C, HP), jnp.int32),
            pltpu.VMEM((4, C), jnp.float32),
        ] + [pltpu.SemaphoreType.DMA] * 12,
    )
    def k(emb_h, wrel_h, idx_h, out_h,
          wrel_v, idx_v, s_v, o_v, out_v,
          semi0, semi1, semi2, semi3, semr0, semr1, semr2, semr3,
          semo0, semo1, semo2, semo3):
        wid = lax.axis_index("s") * NC + lax.axis_index("c")
        base_chunk = wid * npw
        semi = (semi0, semi1, semi2, semi3)
        semr = (semr0, semr1, semr2, semr3)
        semo = (semo0, semo1, semo2, semo3)

        pltpu.sync_copy(wrel_h, wrel_v)

        def issue_idx(t, b):
            pltpu.async_copy(idx_h.at[base_chunk + t], idx_v.at[b], semi[b])

        def wait_idx(b):
            pltpu.make_async_copy(idx_h.at[0], idx_v.at[b], semi[b]).wait()

        def issue_rows(b):
            for h in range(2):
                sl = pl.ds(h * (C // 2), C // 2)
                pltpu.async_copy(emb_h.at[idx_v.at[b, 0, sl]],
                                 s_v.at[b, sl], semr[b])
                pltpu.async_copy(emb_h.at[idx_v.at[b, 2, sl]],
                                 o_v.at[b, sl], semr[b])

        def wait_rows(b):
            for h in range(2):
                sl = pl.ds(h * (C // 2), C // 2)
                pltpu.make_async_copy(emb_h.at[idx_v.at[b, 0, sl]],
                                      s_v.at[b, sl], semr[b]).wait()
                pltpu.make_async_copy(emb_h.at[idx_v.at[b, 2, sl]],
                                      o_v.at[b, sl], semr[b]).wait()

        def store_out(t, b):
            off = (base_chunk + t) * C
            pltpu.async_copy(out_v.at[b], out_h.at[pl.ds(off, C)], semo[b])

        def wait_out(t, b):
            off = (base_chunk + t) * C
            pltpu.make_async_copy(out_v.at[b], out_h.at[pl.ds(off, C)],
                                  semo[b]).wait()

        himask = jnp.full((L,), -65536, jnp.int32)

        def unpack2(x):
            lo = plsc.bitcast(lax.shift_left(x, 16), jnp.float32)
            hi = plsc.bitcast(lax.bitwise_and(x, himask), jnp.float32)
            return lo, hi

        def compute(b):
            @plsc.parallel_loop(0, C // L, 1, unroll=2)
            def iblk(i0):
                rows = i0 * L + lax.iota(jnp.int32, L)
                relv = idx_v[b, 1, pl.ds(i0 * L, L)]
                lane = lax.iota(jnp.int32, L)
                zero = jnp.zeros((L,), jnp.float32)

                def dgrp(g, accs):
                    accs = list(accs)
                    for dd in range(8):
                        d = g * 8 + dd
                        cols = (lane + d) & (HP - 1)
                        sp = plsc.load_gather(s_v.at[b], [rows, cols])
                        op_ = plsc.load_gather(o_v.at[b], [rows, cols])
                        rp = plsc.load_gather(wrel_v, [relv, cols])
                        slo, shi = unpack2(sp)
                        olo, ohi = unpack2(op_)
                        rlo, rhi = unpack2(rp)
                        j = dd % 2
                        accs[j] = accs[j] + slo * olo * rlo
                        accs[2 + j] = accs[2 + j] + shi * ohi * rhi
                    return tuple(accs)

                accs = lax.fori_loop(0, HP // 8, dgrp,
                                     (zero, zero, zero, zero))
                out_v[b, pl.ds(i0 * L, L)] = (
                    (accs[0] + accs[1]) + (accs[2] + accs[3]))

        for j in range(4):
            issue_idx(j, j)
        for j in range(3):
            wait_idx(j)
            issue_rows(j)

        def step(t, b):
            wait_rows(b)

            @pl.when(t + 3 < npw)
            def _():
                wait_idx((b + 3) % 4)
                issue_rows((b + 3) % 4)

            @pl.when(t >= 4)
            def _():
                wait_out(t - 4, b)

            compute(b)
            @pl.when(t + 4 < npw)
            def _():
                issue_idx(t + 4, b)

            store_out(t, b)

        def outer(g, _):
            for j in range(4):
                step(g * 4 + j, j)
            return _

        lax.fori_loop(0, npw // 4, outer, None)
        for j in range(4):
            wait_out(npw - 4 + j, j)

    return k(emb, wrel, idx_all)


def _pack_bf16(table):
    b = table.astype(jnp.bfloat16).reshape(table.shape[0], HP, 2)
    return lax.bitcast_convert_type(b, jnp.int32)


def kernel(embedding, w_relation, triplets):
    n = triplets.shape[0]
    n_chunks = -(-n // C)
    npw = -(-n_chunks // NW)
    n_chunks = NW * npw
    np_total = n_chunks * C
    trip = jnp.pad(triplets.astype(jnp.int32), ((0, np_total - n), (0, 0)))
    idx_all = trip.reshape(n_chunks, C, 3).transpose(0, 2, 1)
    out = _sc_score(_pack_bf16(embedding), _pack_bf16(w_relation),
                    idx_all, npw)
    return out[:n]


# speedup vs baseline: 1.0705x; 1.0280x over previous
import functools

import jax
import jax.numpy as jnp
from jax import lax
from jax.experimental import pallas as pl
from jax.experimental.pallas import tpu as pltpu
from jax.experimental.pallas import tpu_sc as plsc

H = 64
HP = H // 2
C = 128
L = 16
NC = 2
NS = 16
NW = NC * NS
N_REL = 100
NV_PAD = 50048


@functools.partial(jax.jit, static_argnames=("npw",))
def _sc_score(emb, wrel, idx_all, npw):
    n_chunks = idx_all.shape[0]
    np_total = n_chunks * C
    mesh = plsc.VectorSubcoreMesh(core_axis_name="c", subcore_axis_name="s")

    @functools.partial(
        pl.kernel,
        mesh=mesh,
        compiler_params=pltpu.CompilerParams(
            needs_layout_passes=False, use_tc_tiling_on_sc=False),
        out_type=jax.ShapeDtypeStruct((np_total,), jnp.float32),
        scratch_types=[
            pltpu.VMEM_SHARED((NV_PAD, HP), jnp.int32),
            pltpu.VMEM((N_REL, HP), jnp.int32),
            pltpu.VMEM((2, 3, C), jnp.int32),
            pltpu.VMEM((2, C, HP), jnp.int32),
            pltpu.VMEM((2, C, HP), jnp.int32),
            pltpu.VMEM((2, C), jnp.float32),
            pltpu.SemaphoreType.DMA,
            pltpu.SemaphoreType.DMA,
            pltpu.SemaphoreType.DMA,
            pltpu.SemaphoreType.DMA,
            pltpu.SemaphoreType.DMA,
            pltpu.SemaphoreType.DMA,
        ],
    )
    def k(emb_h, wrel_h, idx_h, out_h,
          shared_v, wrel_v, idx_v, s_v, o_v, out_v,
          semi0, semi1, semr0, semr1, semo0, semo1):
        wid = lax.axis_index("s") * NC + lax.axis_index("c")
        base_chunk = wid * npw
        semi = (semi0, semi1)
        semr = (semr0, semr1)
        semo = (semo0, semo1)

        pltpu.sync_copy(wrel_h, wrel_v)

        rpt = NV_PAD // NS
        sid = lax.axis_index("s")
        pltpu.sync_copy(emb_h.at[pl.ds(sid * rpt, rpt)],
                        shared_v.at[pl.ds(sid * rpt, rpt)])
        plsc.subcore_barrier()

        def issue_idx(t, b):
            pltpu.async_copy(idx_h.at[base_chunk + t], idx_v.at[b], semi[b])

        def wait_idx(b):
            pltpu.make_async_copy(idx_h.at[0], idx_v.at[b], semi[b]).wait()

        def issue_rows(b):
            pltpu.async_copy(shared_v.at[idx_v.at[b, 0]], s_v.at[b], semr[b])
            pltpu.async_copy(shared_v.at[idx_v.at[b, 2]], o_v.at[b], semr[b])

        def wait_rows(b):
            pltpu.make_async_copy(shared_v.at[idx_v.at[b, 0]], s_v.at[b],
                                  semr[b]).wait()
            pltpu.make_async_copy(shared_v.at[idx_v.at[b, 2]], o_v.at[b],
                                  semr[b]).wait()

        def store_out(t, b):
            off = (base_chunk + t) * C
            pltpu.async_copy(out_v.at[b], out_h.at[pl.ds(off, C)], semo[b])

        def wait_out(t, b):
            off = (base_chunk + t) * C
            pltpu.make_async_copy(out_v.at[b], out_h.at[pl.ds(off, C)],
                                  semo[b]).wait()

        himask = jnp.full((L,), -65536, jnp.int32)

        def unpack2(x):
            lo = plsc.bitcast(lax.shift_left(x, 16), jnp.float32)
            hi = plsc.bitcast(lax.bitwise_and(x, himask), jnp.float32)
            return lo, hi

        def compute(b):
            @plsc.parallel_loop(0, C // L, 1, unroll=2)
            def iblk(i0):
                rows = i0 * L + lax.iota(jnp.int32, L)
                relv = idx_v[b, 1, pl.ds(i0 * L, L)]
                lane = lax.iota(jnp.int32, L)
                zero = jnp.zeros((L,), jnp.float32)

                def dgrp(g, accs):
                    accs = list(accs)
                    for dd in range(8):
                        d = g * 8 + dd
                        cols = (lane + d) & (HP - 1)
                        sp = plsc.load_gather(s_v.at[b], [rows, cols])
                        op_ = plsc.load_gather(o_v.at[b], [rows, cols])
                        rp = plsc.load_gather(wrel_v, [relv, cols])
                        slo, shi = unpack2(sp)
                        olo, ohi = unpack2(op_)
                        rlo, rhi = unpack2(rp)
                        j = dd % 2
                        accs[j] = accs[j] + slo * olo * rlo
                        accs[2 + j] = accs[2 + j] + shi * ohi * rhi
                    return tuple(accs)

                accs = lax.fori_loop(0, HP // 8, dgrp,
                                     (zero, zero, zero, zero))
                out_v[b, pl.ds(i0 * L, L)] = (
                    (accs[0] + accs[1]) + (accs[2] + accs[3]))

        issue_idx(0, 0)
        issue_idx(1, 1)
        wait_idx(0)
        issue_rows(0)

        def step(t, b):
            wait_rows(b)

            @pl.when(t + 1 < npw)
            def _():
                wait_idx(1 - b)
                issue_rows(1 - b)

            @pl.when(t >= 2)
            def _():
                wait_out(t - 2, b)

            compute(b)
            @pl.when(t + 2 < npw)
            def _():
                issue_idx(t + 2, b)

            store_out(t, b)

        def outer(g, _):
            step(g * 2, 0)
            step(g * 2 + 1, 1)
            return _

        lax.fori_loop(0, npw // 2, outer, None)
        wait_out(npw - 2, 0)
        wait_out(npw - 1, 1)

    return k(emb, wrel, idx_all)


def _pack_bf16(table):
    b = table.astype(jnp.bfloat16).reshape(table.shape[0], HP, 2)
    return lax.bitcast_convert_type(b, jnp.int32)


def kernel(embedding, w_relation, triplets):
    n = triplets.shape[0]
    n_chunks = -(-n // C)
    npw = -(-n_chunks // NW)
    n_chunks = NW * npw
    np_total = n_chunks * C
    trip = jnp.pad(triplets.astype(jnp.int32), ((0, np_total - n), (0, 0)))
    idx_all = trip.reshape(n_chunks, C, 3).transpose(0, 2, 1)
    emb_p = jnp.pad(_pack_bf16(embedding),
                    ((0, NV_PAD - embedding.shape[0]), (0, 0)))
    out = _sc_score(emb_p, _pack_bf16(w_relation), idx_all, npw)
    return out[:n]


# speedup vs baseline: 1.1292x; 1.0548x over previous
import functools

import jax
import jax.numpy as jnp
from jax import lax
from jax.experimental import pallas as pl
from jax.experimental.pallas import tpu as pltpu
from jax.experimental.pallas import tpu_sc as plsc

H = 64
HP = H // 2
C = 128
L = 16
NC = 2
NS = 16
NW = NC * NS
N_REL = 100
NV_PAD = 50048


@functools.partial(jax.jit, static_argnames=("npw",))
def _sc_score(emb, wrel, idx_all, npw):
    n_chunks = idx_all.shape[0]
    np_total = n_chunks * C
    mesh = plsc.VectorSubcoreMesh(core_axis_name="c", subcore_axis_name="s")

    @functools.partial(
        pl.kernel,
        mesh=mesh,
        compiler_params=pltpu.CompilerParams(
            needs_layout_passes=False, use_tc_tiling_on_sc=False),
        out_type=jax.ShapeDtypeStruct((np_total,), jnp.float32),
        scratch_types=[
            pltpu.VMEM_SHARED((NV_PAD, HP), jnp.int32),
            pltpu.VMEM((N_REL, HP), jnp.int32),
            pltpu.VMEM((2, 3, C), jnp.int32),
            pltpu.VMEM((2, C, HP), jnp.int32),
            pltpu.VMEM((2, C, HP), jnp.int32),
            pltpu.VMEM((2, C), jnp.float32),
            pltpu.SemaphoreType.DMA,
            pltpu.SemaphoreType.DMA,
            pltpu.SemaphoreType.DMA,
            pltpu.SemaphoreType.DMA,
            pltpu.SemaphoreType.DMA,
            pltpu.SemaphoreType.DMA,
        ],
    )
    def k(emb_h, wrel_h, idx_h, out_h,
          shared_v, wrel_v, idx_v, s_v, o_v, out_v,
          semi0, semi1, semr0, semr1, semo0, semo1):
        wid = lax.axis_index("s") * NC + lax.axis_index("c")
        base_chunk = wid * npw
        semi = (semi0, semi1)
        semr = (semr0, semr1)
        semo = (semo0, semo1)

        pltpu.sync_copy(wrel_h, wrel_v)

        rpt = NV_PAD // NS
        sid = lax.axis_index("s")
        pltpu.sync_copy(emb_h.at[pl.ds(sid * rpt, rpt)],
                        shared_v.at[pl.ds(sid * rpt, rpt)])
        plsc.subcore_barrier()

        def issue_idx(t, b):
            pltpu.async_copy(idx_h.at[base_chunk + t], idx_v.at[b], semi[b])

        def wait_idx(b):
            pltpu.make_async_copy(idx_h.at[0], idx_v.at[b], semi[b]).wait()

        def issue_rows(b):
            pltpu.async_copy(shared_v.at[idx_v.at[b, 0]], s_v.at[b], semr[b])
            pltpu.async_copy(shared_v.at[idx_v.at[b, 2]], o_v.at[b], semr[b])

        def wait_rows(b):
            pltpu.make_async_copy(shared_v.at[idx_v.at[b, 0]], s_v.at[b],
                                  semr[b]).wait()
            pltpu.make_async_copy(shared_v.at[idx_v.at[b, 2]], o_v.at[b],
                                  semr[b]).wait()

        def store_out(t, b):
            off = (base_chunk + t) * C
            pltpu.async_copy(out_v.at[b], out_h.at[pl.ds(off, C)], semo[b])

        def wait_out(t, b):
            off = (base_chunk + t) * C
            pltpu.make_async_copy(out_v.at[b], out_h.at[pl.ds(off, C)],
                                  semo[b]).wait()

        himask = jnp.full((L,), -65536, jnp.int32)

        def unpack2(x):
            lo = plsc.bitcast(lax.shift_left(x, 16), jnp.float32)
            hi = plsc.bitcast(lax.bitwise_and(x, himask), jnp.float32)
            return lo, hi

        def compute(b):
            @plsc.parallel_loop(0, C // L, 1, unroll=2)
            def iblk(i0):
                rows = i0 * L + lax.iota(jnp.int32, L)
                relv = idx_v[b, 1, pl.ds(i0 * L, L)]
                lane = lax.iota(jnp.int32, L)
                zero = jnp.zeros((L,), jnp.float32)

                def dgrp(g, accs):
                    accs = list(accs)
                    for dd in range(8):
                        d = g * 8 + dd
                        cols = (lane + d) & (HP - 1)
                        sp = plsc.load_gather(s_v.at[b], [rows, cols])
                        op_ = plsc.load_gather(o_v.at[b], [rows, cols])
                        rp = plsc.load_gather(wrel_v, [relv, cols])
                        qb = (plsc.bitcast(sp, jnp.bfloat16) *
                              plsc.bitcast(op_, jnp.bfloat16))
                        qlo, qhi = unpack2(plsc.bitcast(qb, jnp.int32))
                        rlo, rhi = unpack2(rp)
                        j = dd % 2
                        accs[j] = accs[j] + qlo * rlo
                        accs[2 + j] = accs[2 + j] + qhi * rhi
                    return tuple(accs)

                accs = lax.fori_loop(0, HP // 8, dgrp,
                                     (zero, zero, zero, zero))
                out_v[b, pl.ds(i0 * L, L)] = (
                    (accs[0] + accs[1]) + (accs[2] + accs[3]))

        issue_idx(0, 0)
        issue_idx(1, 1)
        wait_idx(0)
        issue_rows(0)

        def step(t, b):
            wait_rows(b)

            @pl.when(t + 1 < npw)
            def _():
                wait_idx(1 - b)
                issue_rows(1 - b)

            @pl.when(t >= 2)
            def _():
                wait_out(t - 2, b)

            compute(b)
            @pl.when(t + 2 < npw)
            def _():
                issue_idx(t + 2, b)

            store_out(t, b)

        def outer(g, _):
            step(g * 2, 0)
            step(g * 2 + 1, 1)
            return _

        lax.fori_loop(0, npw // 2, outer, None)
        wait_out(npw - 2, 0)
        wait_out(npw - 1, 1)

    return k(emb, wrel, idx_all)


def _pack_bf16(table):
    b = table.astype(jnp.bfloat16).reshape(table.shape[0], HP, 2)
    return lax.bitcast_convert_type(b, jnp.int32)


def kernel(embedding, w_relation, triplets):
    n = triplets.shape[0]
    n_chunks = -(-n // C)
    npw = -(-n_chunks // NW)
    n_chunks = NW * npw
    np_total = n_chunks * C
    trip = jnp.pad(triplets.astype(jnp.int32), ((0, np_total - n), (0, 0)))
    idx_all = trip.reshape(n_chunks, C, 3).transpose(0, 2, 1)
    emb_p = jnp.pad(_pack_bf16(embedding),
                    ((0, NV_PAD - embedding.shape[0]), (0, 0)))
    out = _sc_score(emb_p, _pack_bf16(w_relation), idx_all, npw)
    return out[:n]


# speedup vs baseline: 1.2074x; 1.0692x over previous
import functools

import jax
import jax.numpy as jnp
from jax import lax
from jax.experimental import pallas as pl
from jax.experimental.pallas import tpu as pltpu
from jax.experimental.pallas import tpu_sc as plsc

H = 64
HP = H // 2
C = 128
L = 16
NC = 2
NS = 16
NW = NC * NS
N_REL = 100
NV_PAD = 50048


@functools.partial(jax.jit, static_argnames=("npw",))
def _sc_score(emb, wrel, idx_all, npw):
    n_chunks = idx_all.shape[0]
    np_total = n_chunks * C
    mesh = plsc.VectorSubcoreMesh(core_axis_name="c", subcore_axis_name="s")

    @functools.partial(
        pl.kernel,
        mesh=mesh,
        compiler_params=pltpu.CompilerParams(
            needs_layout_passes=False, use_tc_tiling_on_sc=False),
        out_type=jax.ShapeDtypeStruct((np_total,), jnp.float32),
        scratch_types=[
            pltpu.VMEM_SHARED((NV_PAD, HP), jnp.int32),
            pltpu.VMEM((N_REL, HP), jnp.int32),
            pltpu.VMEM((2, 3, C), jnp.int32),
            pltpu.VMEM((2, C, HP), jnp.int32),
            pltpu.VMEM((2, C, HP), jnp.int32),
            pltpu.VMEM((2, C), jnp.float32),
            pltpu.SemaphoreType.DMA,
            pltpu.SemaphoreType.DMA,
            pltpu.SemaphoreType.DMA,
            pltpu.SemaphoreType.DMA,
            pltpu.SemaphoreType.DMA,
            pltpu.SemaphoreType.DMA,
        ],
    )
    def k(emb_h, wrel_h, idx_h, out_h,
          shared_v, wrel_v, idx_v, s_v, o_v, out_v,
          semi0, semi1, semr0, semr1, semo0, semo1):
        wid = lax.axis_index("s") * NC + lax.axis_index("c")
        base_chunk = wid * npw
        semi = (semi0, semi1)
        semr = (semr0, semr1)
        semo = (semo0, semo1)

        pltpu.sync_copy(wrel_h, wrel_v)

        rpt = NV_PAD // NS
        sid = lax.axis_index("s")
        pltpu.sync_copy(emb_h.at[pl.ds(sid * rpt, rpt)],
                        shared_v.at[pl.ds(sid * rpt, rpt)])
        plsc.subcore_barrier()

        def issue_idx(t, b):
            pltpu.async_copy(idx_h.at[base_chunk + t], idx_v.at[b], semi[b])

        def wait_idx(b):
            pltpu.make_async_copy(idx_h.at[0], idx_v.at[b], semi[b]).wait()

        def issue_rows(b):
            pltpu.async_copy(shared_v.at[idx_v.at[b, 0]], s_v.at[b], semr[b])
            pltpu.async_copy(shared_v.at[idx_v.at[b, 2]], o_v.at[b], semr[b])

        def wait_rows(b):
            pltpu.make_async_copy(shared_v.at[idx_v.at[b, 0]], s_v.at[b],
                                  semr[b]).wait()
            pltpu.make_async_copy(shared_v.at[idx_v.at[b, 2]], o_v.at[b],
                                  semr[b]).wait()

        def store_out(t, b):
            off = (base_chunk + t) * C
            pltpu.async_copy(out_v.at[b], out_h.at[pl.ds(off, C)], semo[b])

        def wait_out(t, b):
            off = (base_chunk + t) * C
            pltpu.make_async_copy(out_v.at[b], out_h.at[pl.ds(off, C)],
                                  semo[b]).wait()

        himask = jnp.full((L,), -65536, jnp.int32)

        def unpack2(x):
            lo = plsc.bitcast(lax.shift_left(x, 16), jnp.float32)
            hi = plsc.bitcast(lax.bitwise_and(x, himask), jnp.float32)
            return lo, hi

        def compute(b):
            @plsc.parallel_loop(0, C // L, 1, unroll=2)
            def iblk(i0):
                rows = i0 * L + lax.iota(jnp.int32, L)
                relv = idx_v[b, 1, pl.ds(i0 * L, L)]
                lane = lax.iota(jnp.int32, L)
                zero = jnp.zeros((L,), jnp.float32)

                def dgrp(g, accs):
                    accs = list(accs)
                    for dd in range(8):
                        d = g * 8 + dd
                        cols = (lane + d) & (HP - 1)
                        sp = plsc.load_gather(s_v.at[b], [rows, cols])
                        op_ = plsc.load_gather(o_v.at[b], [rows, cols])
                        rp = plsc.load_gather(wrel_v, [relv, cols])
                        qb = (plsc.bitcast(sp, jnp.bfloat16) *
                              plsc.bitcast(op_, jnp.bfloat16) *
                              plsc.bitcast(rp, jnp.bfloat16))
                        qlo, qhi = unpack2(plsc.bitcast(qb, jnp.int32))
                        j = dd % 2
                        accs[j] = accs[j] + qlo
                        accs[2 + j] = accs[2 + j] + qhi
                    return tuple(accs)

                accs = lax.fori_loop(0, HP // 8, dgrp,
                                     (zero, zero, zero, zero))
                out_v[b, pl.ds(i0 * L, L)] = (
                    (accs[0] + accs[1]) + (accs[2] + accs[3]))

        issue_idx(0, 0)
        issue_idx(1, 1)
        wait_idx(0)
        issue_rows(0)

        def step(t, b):
            wait_rows(b)

            @pl.when(t + 1 < npw)
            def _():
                wait_idx(1 - b)
                issue_rows(1 - b)

            @pl.when(t >= 2)
            def _():
                wait_out(t - 2, b)

            compute(b)
            @pl.when(t + 2 < npw)
            def _():
                issue_idx(t + 2, b)

            store_out(t, b)

        def outer(g, _):
            step(g * 2, 0)
            step(g * 2 + 1, 1)
            return _

        lax.fori_loop(0, npw // 2, outer, None)
        wait_out(npw - 2, 0)
        wait_out(npw - 1, 1)

    return k(emb, wrel, idx_all)


def _pack_bf16(table):
    b = table.astype(jnp.bfloat16).reshape(table.shape[0], HP, 2)
    return lax.bitcast_convert_type(b, jnp.int32)


def kernel(embedding, w_relation, triplets):
    n = triplets.shape[0]
    n_chunks = -(-n // C)
    npw = -(-n_chunks // NW)
    n_chunks = NW * npw
    np_total = n_chunks * C
    trip = jnp.pad(triplets.astype(jnp.int32), ((0, np_total - n), (0, 0)))
    idx_all = trip.reshape(n_chunks, C, 3).transpose(0, 2, 1)
    emb_p = jnp.pad(_pack_bf16(embedding),
                    ((0, NV_PAD - embedding.shape[0]), (0, 0)))
    out = _sc_score(emb_p, _pack_bf16(w_relation), idx_all, npw)
    return out[:n]
